# baseline (speedup 1.0000x reference)
import jax
import jax.numpy as jnp
import numpy as np
from jax import lax
from jax.experimental import pallas as pl
from jax.experimental.pallas import tpu as pltpu
from jax.experimental.pallas import tpu_sc as plsc

COORD_DIM = 3
N_LEVELS = 16
F = 2
LOG2_T = 19
T = 1 << LOG2_T
N_MIN = 16
N_MAX = 512
BSZ = 131072
FACTOR = np.exp((np.log(N_MAX) - np.log(N_MIN)) / (N_LEVELS - 1))
P1 = np.int32(np.uint32(2654435761).astype(np.int64) - (1 << 32))
P2 = np.int32(805459861)

_RES = [float(np.floor(N_MIN * FACTOR**i)) for i in range(N_LEVELS)]
_BOX = [np.float32(2.0 / r) for r in _RES]
_RESM1 = [np.float32(r - 1.0) for r in _RES]

NC = 2
NS = 16
NW = NC * NS
PW = BSZ // NW
PC = 512
NCHUNK = PW // PC
NG = PC // 16
NIDX = PC * 8 * F
OD = N_LEVELS * F
SLAB = T * F


def _body(x_hbm, tab_hbm, out_hbm, x_v, idx_v, rows_v, w_v, out_v, spm, gsem,
          ssem):
    cid = lax.axis_index("c")
    sid = lax.axis_index("s")
    wid = sid * NC + cid
    wbase = wid * PW

    pltpu.sync_copy(x_hbm.at[pl.ds(wbase * COORD_DIM, PW * COORD_DIM)], x_v)

    iota = lax.iota(jnp.int32, 16)

    def make_phase_a(l):
        box_s = jnp.full((16,), _BOX[l], jnp.float32)
        resm1_s = jnp.full((16,), _RESM1[l], jnp.float32)

        def phase_a(cc):
            par = cc & 1
            cbase = cc * PC

            def ga(g, c3):
                o = cbase + g * 16
                pos3 = (o + iota) * COORD_DIM
                hs = []
                ws = []
                for d in range(COORD_DIM):
                    xd = plsc.load_gather(x_v, [pos3 + d])
                    t = (xd + 1.0) / box_s
                    tc = jnp.minimum(jnp.maximum(t, 0.0), resm1_s)
                    bl = tc.astype(jnp.int32)
                    w = t - bl.astype(jnp.float32)
                    ws.append(w)
                    if d == 0:
                        hs.append((bl, bl + 1))
                    elif d == 1:
                        m = bl * P1
                        hs.append((m, m + P1))
                    else:
                        m = bl * P2
                        hs.append((m, m + P2))
                for d in range(COORD_DIM):
                    w_v[pl.ds((par * COORD_DIM + d) * (NG * 16) + g * 16, 16)] = ws[d]
                for c in range(8):
                    h = hs[0][c & 1] ^ hs[1][(c >> 1) & 1] ^ hs[2][(c >> 2) & 1]
                    h = h & (T - 1)
                    a0 = ((h >> 7) << 8) + (h & 127)
                    base = par * NIDX + g * 256 + c * 32
                    idx_v[pl.ds(base, 16)] = a0
                    idx_v[pl.ds(base + 16, 16)] = a0 + 128
                return c3

            lax.fori_loop(0, NG, ga, 0)
            pltpu.async_copy(
                spm.at[idx_v.at[pl.ds(par * NIDX, NIDX)]],
                rows_v.at[pl.ds(par * NIDX, NIDX)],
                gsem,
            )

        return phase_a

    def gather_wait(cc):
        par = cc & 1
        pltpu.make_async_copy(
            spm.at[idx_v.at[pl.ds(par * NIDX, NIDX)]],
            rows_v.at[pl.ds(par * NIDX, NIDX)],
            gsem,
        ).wait()

    def make_phase_b(l):
        cbq = l >> 2
        ci0 = (2 * l) & 7

        def phase_b(cc):
            par = cc & 1
            cbase = cc * PC

            def gb(g, c3):
                w0 = w_v[pl.ds((par * COORD_DIM + 0) * (NG * 16) + g * 16, 16)]
                w1 = w_v[pl.ds((par * COORD_DIM + 1) * (NG * 16) + g * 16, 16)]
                w2 = w_v[pl.ds((par * COORD_DIM + 2) * (NG * 16) + g * 16, 16)]
                u0 = 1.0 - w0
                u1 = 1.0 - w1
                u2 = 1.0 - w2
                sel = [(u0, w0), (u1, w1), (u2, w2)]
                o2 = par * NIDX + g * 256
                acc0 = None
                acc1 = None
                for c in range(8):
                    cw = (sel[0][c & 1] * sel[1][(c >> 1) & 1]) * sel[2][(c >> 2) & 1]
                    e0 = rows_v[pl.ds(o2 + c * 32, 16)]
                    e1 = rows_v[pl.ds(o2 + c * 32 + 16, 16)]
                    if acc0 is None:
                        acc0 = cw * e0
                        acc1 = cw * e1
                    else:
                        acc0 = acc0 + cw * e0
                        acc1 = acc1 + cw * e1
                ob = (g >> 3) * 256 + (g & 7) * 16
                out_v[pl.ds(ob, 16)] = acc0
                out_v[pl.ds(ob + 128, 16)] = acc1
                return c3

            lax.fori_loop(0, NG, gb, 0)

            for pb in range(PC // 128):
                pltpu.sync_copy(
                    out_v.at[pl.ds(pb * 256, 256)],
                    out_hbm.at[
                        pl.ds(
                            cbq * (BSZ * 8)
                            + (wbase + cbase + pb * 128) * 8
                            + ci0 * 128,
                            256,
                        )
                    ],
                )

        return phase_b

    stg = SLAB // NS
    sbase = sid * stg

    for l in range(N_LEVELS):
        plsc.subcore_barrier()
        pltpu.async_copy(
            tab_hbm.at[pl.ds(l * SLAB + sbase, stg)],
            spm.at[pl.ds(sbase, stg)],
            ssem,
        ).wait()
        plsc.subcore_barrier()

        phase_a = make_phase_a(l)
        phase_b = make_phase_b(l)

        phase_a(0)

        def chunk_body(cc, carry, phase_a=phase_a, phase_b=phase_b):
            phase_a(cc)
            gather_wait(cc - 1)
            phase_b(cc - 1)
            return carry

        lax.fori_loop(1, NCHUNK, chunk_body, 0)
        gather_wait(NCHUNK - 1)
        phase_b(NCHUNK - 1)


@jax.jit
def kernel(x, tables):
    xf = x.reshape(-1)
    tabf = tables.reshape(N_LEVELS, T // 128, 128, F).transpose(0, 1, 3, 2).reshape(-1)

    mesh = plsc.VectorSubcoreMesh(
        core_axis_name="c", subcore_axis_name="s", num_cores=NC, num_subcores=NS
    )
    call = pl.kernel(
        _body,
        out_type=jax.ShapeDtypeStruct((BSZ * OD,), jnp.float32),
        mesh=mesh,
        compiler_params=pltpu.CompilerParams(
            needs_layout_passes=False, use_tc_tiling_on_sc=False
        ),
        scratch_types=[
            pltpu.VMEM((COORD_DIM * PW,), jnp.float32),
            pltpu.VMEM((2 * NIDX,), jnp.int32),
            pltpu.VMEM((2 * NIDX,), jnp.float32),
            pltpu.VMEM((2 * COORD_DIM * NG * 16,), jnp.float32),
            pltpu.VMEM((F * PC,), jnp.float32),
            pltpu.VMEM_SHARED((SLAB,), jnp.float32),
            pltpu.SemaphoreType.DMA,
            pltpu.SemaphoreType.DMA,
        ],
    )
    out = call(xf, tabf)
    out = out.reshape(OD // 8, BSZ // 128, 8, 128)
    return out.transpose(1, 3, 0, 2).reshape(BSZ, OD)


# speedup vs baseline: 1.0036x; 1.0036x over previous
import jax
import jax.numpy as jnp
import numpy as np
from jax import lax
from jax.experimental import pallas as pl
from jax.experimental.pallas import tpu as pltpu
from jax.experimental.pallas import tpu_sc as plsc

COORD_DIM = 3
N_LEVELS = 16
F = 2
LOG2_T = 19
T = 1 << LOG2_T
N_MIN = 16
N_MAX = 512
BSZ = 131072
FACTOR = np.exp((np.log(N_MAX) - np.log(N_MIN)) / (N_LEVELS - 1))
P1 = np.int32(np.uint32(2654435761).astype(np.int64) - (1 << 32))
P2 = np.int32(805459861)

_RES = [float(np.floor(N_MIN * FACTOR**i)) for i in range(N_LEVELS)]
_BOX = [np.float32(2.0 / r) for r in _RES]
_RESM1 = [np.float32(r - 1.0) for r in _RES]

NC = 2
NS = 16
NW = NC * NS
PW = BSZ // NW
PC = 512
NCHUNK = PW // PC
NG = PC // 16
NIDX = PC * 8 * F
OD = N_LEVELS * F
SLAB = T * F


def _body(x_hbm, tab_hbm, out_hbm, x_v, idx_v, rows_v, w_v, out_v, spm, gsem,
          gsem2, ssem):
    cid = lax.axis_index("c")
    sid = lax.axis_index("s")
    wid = sid * NC + cid
    wbase = wid * PW

    pltpu.sync_copy(x_hbm.at[pl.ds(wbase * COORD_DIM, PW * COORD_DIM)], x_v)

    iota = lax.iota(jnp.int32, 16)

    def make_phase_a(l):
        box_s = jnp.full((16,), _BOX[l], jnp.float32)
        resm1_s = jnp.full((16,), _RESM1[l], jnp.float32)

        def phase_a(cc):
            par = cc & 1
            cbase = cc * PC

            def ga(g, c3):
                o = cbase + g * 16
                pos3 = (o + iota) * COORD_DIM
                hs = []
                ws = []
                for d in range(COORD_DIM):
                    xd = plsc.load_gather(x_v, [pos3 + d])
                    t = (xd + 1.0) / box_s
                    tc = jnp.minimum(jnp.maximum(t, 0.0), resm1_s)
                    bl = tc.astype(jnp.int32)
                    w = t - bl.astype(jnp.float32)
                    ws.append(w)
                    if d == 0:
                        hs.append((bl, bl + 1))
                    elif d == 1:
                        m = bl * P1
                        hs.append((m, m + P1))
                    else:
                        m = bl * P2
                        hs.append((m, m + P2))
                for d in range(COORD_DIM):
                    w_v[pl.ds((par * COORD_DIM + d) * (NG * 16) + g * 16, 16)] = ws[d]
                for c in range(8):
                    h = hs[0][c & 1] ^ hs[1][(c >> 1) & 1] ^ hs[2][(c >> 2) & 1]
                    h = h & (T - 1)
                    a0 = ((h >> 7) << 8) + (h & 127)
                    base = par * NIDX + g * 256 + c * 32
                    idx_v[pl.ds(base, 16)] = a0
                    idx_v[pl.ds(base + 16, 16)] = a0 + 128
                return c3

            lax.fori_loop(0, NG, ga, 0)
            h = NIDX // 2
            pltpu.async_copy(
                spm.at[idx_v.at[pl.ds(par * NIDX, h)]],
                rows_v.at[pl.ds(par * NIDX, h)],
                gsem,
            )
            pltpu.async_copy(
                spm.at[idx_v.at[pl.ds(par * NIDX + h, h)]],
                rows_v.at[pl.ds(par * NIDX + h, h)],
                gsem2,
            )

        return phase_a

    def gather_wait(cc):
        par = cc & 1
        h = NIDX // 2
        pltpu.make_async_copy(
            spm.at[idx_v.at[pl.ds(par * NIDX, h)]],
            rows_v.at[pl.ds(par * NIDX, h)],
            gsem,
        ).wait()
        pltpu.make_async_copy(
            spm.at[idx_v.at[pl.ds(par * NIDX + h, h)]],
            rows_v.at[pl.ds(par * NIDX + h, h)],
            gsem2,
        ).wait()

    def make_phase_b(l):
        cbq = l >> 2
        ci0 = (2 * l) & 7

        def phase_b(cc):
            par = cc & 1
            cbase = cc * PC

            def gb(g, c3):
                w0 = w_v[pl.ds((par * COORD_DIM + 0) * (NG * 16) + g * 16, 16)]
                w1 = w_v[pl.ds((par * COORD_DIM + 1) * (NG * 16) + g * 16, 16)]
                w2 = w_v[pl.ds((par * COORD_DIM + 2) * (NG * 16) + g * 16, 16)]
                u0 = 1.0 - w0
                u1 = 1.0 - w1
                u2 = 1.0 - w2
                sel = [(u0, w0), (u1, w1), (u2, w2)]
                o2 = par * NIDX + g * 256
                acc0 = None
                acc1 = None
                for c in range(8):
                    cw = (sel[0][c & 1] * sel[1][(c >> 1) & 1]) * sel[2][(c >> 2) & 1]
                    e0 = rows_v[pl.ds(o2 + c * 32, 16)]
                    e1 = rows_v[pl.ds(o2 + c * 32 + 16, 16)]
                    if acc0 is None:
                        acc0 = cw * e0
                        acc1 = cw * e1
                    else:
                        acc0 = acc0 + cw * e0
                        acc1 = acc1 + cw * e1
                ob = (g >> 3) * 256 + (g & 7) * 16
                out_v[pl.ds(ob, 16)] = acc0
                out_v[pl.ds(ob + 128, 16)] = acc1
                return c3

            lax.fori_loop(0, NG, gb, 0)

            for pb in range(PC // 128):
                pltpu.sync_copy(
                    out_v.at[pl.ds(pb * 256, 256)],
                    out_hbm.at[
                        pl.ds(
                            cbq * (BSZ * 8)
                            + (wbase + cbase + pb * 128) * 8
                            + ci0 * 128,
                            256,
                        )
                    ],
                )

        return phase_b

    stg = SLAB // NS
    sbase = sid * stg

    for l in range(N_LEVELS):
        plsc.subcore_barrier()
        pltpu.async_copy(
            tab_hbm.at[pl.ds(l * SLAB + sbase, stg)],
            spm.at[pl.ds(sbase, stg)],
            ssem,
        ).wait()
        plsc.subcore_barrier()

        phase_a = make_phase_a(l)
        phase_b = make_phase_b(l)

        phase_a(0)

        def chunk_body(cc, carry, phase_a=phase_a, phase_b=phase_b):
            phase_a(cc)
            gather_wait(cc - 1)
            phase_b(cc - 1)
            return carry

        lax.fori_loop(1, NCHUNK, chunk_body, 0)
        gather_wait(NCHUNK - 1)
        phase_b(NCHUNK - 1)


@jax.jit
def kernel(x, tables):
    xf = x.reshape(-1)
    tabf = tables.reshape(N_LEVELS, T // 128, 128, F).transpose(0, 1, 3, 2).reshape(-1)

    mesh = plsc.VectorSubcoreMesh(
        core_axis_name="c", subcore_axis_name="s", num_cores=NC, num_subcores=NS
    )
    call = pl.kernel(
        _body,
        out_type=jax.ShapeDtypeStruct((BSZ * OD,), jnp.float32),
        mesh=mesh,
        compiler_params=pltpu.CompilerParams(
            needs_layout_passes=False, use_tc_tiling_on_sc=False
        ),
        scratch_types=[
            pltpu.VMEM((COORD_DIM * PW,), jnp.float32),
            pltpu.VMEM((2 * NIDX,), jnp.int32),
            pltpu.VMEM((2 * NIDX,), jnp.float32),
            pltpu.VMEM((2 * COORD_DIM * NG * 16,), jnp.float32),
            pltpu.VMEM((F * PC,), jnp.float32),
            pltpu.VMEM_SHARED((SLAB,), jnp.float32),
            pltpu.SemaphoreType.DMA,
            pltpu.SemaphoreType.DMA,
            pltpu.SemaphoreType.DMA,
        ],
    )
    out = call(xf, tabf)
    out = out.reshape(OD // 8, BSZ // 128, 8, 128)
    return out.transpose(1, 3, 0, 2).reshape(BSZ, OD)
